# Initial kernel scaffold; baseline (speedup 1.0000x reference)
import jax
import jax.numpy as jnp
from jax.experimental import pallas as pl


def kernel(x, group, centers, scales):
    raise NotImplementedError("write your pallas kernel here")


# speedup vs baseline: 1485.7615x; 1485.7615x over previous
import functools

import jax
import jax.numpy as jnp
from jax import lax
from jax.experimental import pallas as pl
from jax.experimental.pallas import tpu as pltpu
from jax.experimental.pallas import tpu_sc as plsc

N = 3276800
TBL = 128
NC, NS, L = 2, 16, 16
NW = NC * NS
PER_W = N // NW
CHUNK = 12800
NCHUNK = PER_W // CHUNK


def _body(x_hbm, g_hbm, c_hbm, s_hbm, out_hbm,
          xb0, xb1, gb0, gb1, ob0, ob1, cb, invb, sem_in, sem_out):
    xbs, gbs, obs = (xb0, xb1), (gb0, gb1), (ob0, ob1)
    wid = lax.axis_index("s") * NC + lax.axis_index("c")
    base = wid * PER_W

    def start_in(k):
        off = base + k * CHUNK
        b = k % 2
        pltpu.async_copy(x_hbm.at[pl.ds(off, CHUNK)], xbs[b], sem_in.at[b])
        pltpu.async_copy(g_hbm.at[pl.ds(off, CHUNK)], gbs[b], sem_in.at[b])

    def wait_in(k):
        off = base + k * CHUNK
        b = k % 2
        pltpu.make_async_copy(x_hbm.at[pl.ds(off, CHUNK)], xbs[b],
                              sem_in.at[b]).wait()
        pltpu.make_async_copy(g_hbm.at[pl.ds(off, CHUNK)], gbs[b],
                              sem_in.at[b]).wait()

    def start_out(k):
        off = base + k * CHUNK
        b = k % 2
        pltpu.async_copy(obs[b], out_hbm.at[pl.ds(off, CHUNK)], sem_out.at[b])

    def wait_out(k):
        off = base + k * CHUNK
        b = k % 2
        pltpu.make_async_copy(obs[b], out_hbm.at[pl.ds(off, CHUNK)],
                              sem_out.at[b]).wait()

    start_in(0)

    pltpu.sync_copy(c_hbm, cb)
    pltpu.sync_copy(s_hbm, invb)

    def inv_one(i, _):
        sv = invb[pl.ds(i * L, L)]
        invb[pl.ds(i * L, L)] = 1.0 / sv
        return _
    lax.fori_loop(0, TBL // L, inv_one, None)

    for k in range(NCHUNK):
        b = k % 2
        if k + 1 < NCHUNK:
            start_in(k + 1)
        wait_in(k)
        if k >= 2:
            wait_out(k - 2)
        xk, gk, ok = xbs[b], gbs[b], obs[b]

        @plsc.parallel_loop(0, CHUNK, L, unroll=8)
        def per_vec(i):
            idx = gk[pl.ds(i, L)] - 1
            c = plsc.load_gather(cb, [idx])
            inv = plsc.load_gather(invb, [idx])
            ok[pl.ds(i, L)] = (xk[pl.ds(i, L)] - c) * inv

        start_out(k)

    wait_out(NCHUNK - 2)
    wait_out(NCHUNK - 1)


@jax.jit
def _standardize(x, group, c_pad, s_pad):
    run = functools.partial(
        pl.kernel,
        mesh=plsc.VectorSubcoreMesh(core_axis_name="c", subcore_axis_name="s"),
        out_type=jax.ShapeDtypeStruct((N,), jnp.float32),
        compiler_params=pltpu.CompilerParams(needs_layout_passes=False),
        scratch_types=[
            pltpu.VMEM((CHUNK,), jnp.float32),
            pltpu.VMEM((CHUNK,), jnp.float32),
            pltpu.VMEM((CHUNK,), jnp.int32),
            pltpu.VMEM((CHUNK,), jnp.int32),
            pltpu.VMEM((CHUNK,), jnp.float32),
            pltpu.VMEM((CHUNK,), jnp.float32),
            pltpu.VMEM((TBL,), jnp.float32),
            pltpu.VMEM((TBL,), jnp.float32),
            pltpu.SemaphoreType.DMA((2,)),
            pltpu.SemaphoreType.DMA((2,)),
        ],
    )(_body)
    return run(x, group, c_pad, s_pad)


def kernel(x, group, centers, scales):
    g = centers.shape[0]
    c_pad = jnp.zeros((TBL,), jnp.float32).at[:g].set(centers)
    s_pad = jnp.ones((TBL,), jnp.float32).at[:g].set(scales)
    return _standardize(x, group, c_pad, s_pad)
